# baseline (speedup 1.0000x reference)
import jax
import jax.numpy as jnp
from jax import lax
from jax.experimental import pallas as pl
from jax.experimental.pallas import tpu as pltpu
from jax.experimental.pallas import tpu_sc as plsc

N_FIELDS = 26
EMBED_DIM = 16
FIELD_SIZE = 100000
BATCH = 16384
L = 16
NC, NS = 2, 16
NW = NC * NS
ROWS_W = BATCH // NW
CH = 16
NCHUNK = ROWS_W // CH
NIDX = CH * N_FIELDS
GROUP = 128


def _fm_body(idx_hbm, grp_hbm, emb_hbm, fc_hbm, bias_hbm, out_hbm,
             idx_v, grp_v, emb_v, fc_v, z_v, bias_v, sem_e, sem_f):
    wid = lax.axis_index("s") * NC + lax.axis_index("c")
    pltpu.sync_copy(bias_hbm, bias_v)
    b0 = bias_v[...][0]
    lane = lax.iota(jnp.int32, L)
    mtail = lane < (N_FIELDS - L)

    def chunk(c, _):
        base = wid * (ROWS_W * N_FIELDS) + c * NIDX
        pltpu.sync_copy(idx_hbm.at[pl.ds(base, NIDX)],
                        idx_v.at[pl.ds(0, NIDX)])
        pltpu.sync_copy(grp_hbm.at[pl.ds(base, NIDX)], grp_v)
        cp_e = pltpu.async_copy(emb_hbm.at[grp_v], emb_v, sem_e)
        cp_f = pltpu.async_copy(fc_hbm.at[idx_v.at[pl.ds(0, NIDX)]],
                                fc_v.at[pl.ds(0, NIDX)], sem_f)
        cp_e.wait()
        cp_f.wait()

        def rowi(i, vec):
            rb = i * N_FIELDS
            ia = idx_v[pl.ds(rb, L)]
            ib = idx_v[pl.ds(rb + L, L)]
            ca = (ia & 7) * EMBED_DIM
            cb = (ib & 7) * EMBED_DIM
            acc = emb_v[rb, pl.ds(ca[0], EMBED_DIM)]
            acc2 = acc * acc
            for f in range(1, N_FIELDS):
                col = ca[f] if f < L else cb[f - L]
                v = emb_v[rb + f, pl.ds(col, EMBED_DIM)]
                acc = acc + v
                acc2 = acc2 + v * v
            a = fc_v[pl.ds(rb, L)]
            b = fc_v[pl.ds(rb + L, L)]
            lin = b0 + jnp.sum(a) + jnp.sum(jnp.where(mtail, b, 0.0))
            z = lin + 0.5 * (jnp.sum(acc * acc) - jnp.sum(acc2))
            return jnp.where(lane == i, z, vec)

        vec = lax.fori_loop(0, CH, rowi, jnp.zeros((L,), jnp.float32))
        z_v[...] = 1.0 / (1.0 + jnp.exp(-vec))
        pltpu.sync_copy(z_v, out_hbm.at[pl.ds(wid * ROWS_W + c * CH, CH)])
        return 0

    lax.fori_loop(0, NCHUNK, chunk, 0)


def kernel(x, emb_table, fc_table, bias):
    offsets = jnp.arange(N_FIELDS, dtype=x.dtype) * FIELD_SIZE
    idx = (x + offsets[None, :]).astype(jnp.int32).reshape(-1)
    grp = idx >> 3
    emb_g = emb_table.reshape(-1, GROUP)
    fc_flat = fc_table.reshape(-1)
    bias_pad = jnp.broadcast_to(bias.astype(jnp.float32), (L,))
    mesh = plsc.VectorSubcoreMesh(core_axis_name="c", subcore_axis_name="s")
    fm = pl.kernel(
        _fm_body,
        out_type=jax.ShapeDtypeStruct((BATCH,), jnp.float32),
        mesh=mesh,
        compiler_params=pltpu.CompilerParams(needs_layout_passes=False,
                                             use_tc_tiling_on_sc=True),
        scratch_types=[
            pltpu.VMEM((NIDX + L,), jnp.int32),
            pltpu.VMEM((NIDX,), jnp.int32),
            pltpu.VMEM((NIDX, GROUP), jnp.float32),
            pltpu.VMEM((NIDX + L,), jnp.float32),
            pltpu.VMEM((L,), jnp.float32),
            pltpu.VMEM((L,), jnp.float32),
            pltpu.SemaphoreType.DMA,
            pltpu.SemaphoreType.DMA,
        ],
    )
    return fm(idx, grp, emb_g, fc_flat, bias_pad)


# speedup vs baseline: 2.0550x; 2.0550x over previous
import jax
import jax.numpy as jnp
from jax import lax
from jax.experimental import pallas as pl
from jax.experimental.pallas import tpu as pltpu
from jax.experimental.pallas import tpu_sc as plsc

N_FIELDS = 26
EMBED_DIM = 16
FIELD_SIZE = 100000
BATCH = 16384
TOTAL = 2600000
L = 16
NC, NS = 2, 16
NW = NC * NS
ROWS_W = BATCH // NW
CH = 16
NCHUNK = ROWS_W // CH
NIDX = CH * N_FIELDS
GROUP = 128
NGRP = TOTAL // 8

SEG = 1024
NSEG = TOTAL // SEG
TAIL = TOTAL - NSEG * SEG
SEG_G = SEG // 8


def _tr_body(src_hbm, out_hbm, in_v, out_v, sem_i, sem_o):
    wid = lax.axis_index("s") * NC + lax.axis_index("c")
    iota = lax.iota(jnp.int32, L)
    rowpre = iota >> 3

    def do_block(r0, g0, width):
        cps = [pltpu.async_copy(src_hbm.at[d, pl.ds(r0, width)],
                                in_v.at[d, pl.ds(0, width)], sem_i)
               for d in range(L)]
        for cp in cps:
            cp.wait()
        nvec = width // L

        def rblk(i, _):
            rowbase = i * 2
            for d in range(L):
                vals = in_v[d, pl.ds(i * L, L)]
                rows = rowpre + rowbase
                cols = (iota & 7) * L + d
                plsc.store_scatter(out_v, [rows, cols], vals)
            return 0

        lax.fori_loop(0, nvec, rblk, 0)
        pltpu.async_copy(out_v.at[pl.ds(0, width // 8)],
                         out_hbm.at[pl.ds(g0, width // 8)], sem_o).wait()

    n_w = 79 + jnp.where(wid < NSEG - 79 * NW, 1, 0)

    def seg_loop(i, _):
        seg = i * NW + wid
        do_block(seg * SEG, seg * SEG_G, SEG)
        return 0

    lax.fori_loop(0, n_w, seg_loop, 0)

    @pl.when(wid == NW - 1)
    def _():
        do_block(NSEG * SEG, NSEG * SEG_G, TAIL)


def _fm_body(idx_hbm, grp_hbm, emb_hbm, fc_hbm, bias_hbm, out_hbm,
             idx_v, grp_v, emb_v, fc_v, z_v, bias_v, sem_e, sem_f):
    wid = lax.axis_index("s") * NC + lax.axis_index("c")
    pltpu.sync_copy(bias_hbm, bias_v)
    b0 = bias_v[...][0]
    lane = lax.iota(jnp.int32, L)
    mtail = lane < (N_FIELDS - L)

    def chunk(c, _):
        base = wid * (ROWS_W * N_FIELDS) + c * NIDX
        pltpu.sync_copy(idx_hbm.at[pl.ds(base, NIDX)],
                        idx_v.at[pl.ds(0, NIDX)])
        pltpu.sync_copy(grp_hbm.at[pl.ds(base, NIDX)], grp_v)
        cp_e = pltpu.async_copy(emb_hbm.at[grp_v], emb_v, sem_e)
        cp_f = pltpu.async_copy(fc_hbm.at[idx_v.at[pl.ds(0, NIDX)]],
                                fc_v.at[pl.ds(0, NIDX)], sem_f)
        cp_e.wait()
        cp_f.wait()

        def rowi(i, vec):
            rb = i * N_FIELDS
            ia = idx_v[pl.ds(rb, L)]
            ib = idx_v[pl.ds(rb + L, L)]
            ca = (ia & 7) * EMBED_DIM
            cb = (ib & 7) * EMBED_DIM
            acc = emb_v[rb, pl.ds(ca[0], EMBED_DIM)]
            acc2 = acc * acc
            for f in range(1, N_FIELDS):
                col = ca[f] if f < L else cb[f - L]
                v = emb_v[rb + f, pl.ds(col, EMBED_DIM)]
                acc = acc + v
                acc2 = acc2 + v * v
            a = fc_v[pl.ds(rb, L)]
            b = fc_v[pl.ds(rb + L, L)]
            lin = b0 + jnp.sum(a) + jnp.sum(jnp.where(mtail, b, 0.0))
            z = lin + 0.5 * (jnp.sum(acc * acc) - jnp.sum(acc2))
            return jnp.where(lane == i, z, vec)

        vec = lax.fori_loop(0, CH, rowi, jnp.zeros((L,), jnp.float32))
        z_v[...] = 1.0 / (1.0 + jnp.exp(-vec))
        pltpu.sync_copy(z_v, out_hbm.at[pl.ds(wid * ROWS_W + c * CH, CH)])
        return 0

    lax.fori_loop(0, NCHUNK, chunk, 0)


def kernel(x, emb_table, fc_table, bias):
    offsets = jnp.arange(N_FIELDS, dtype=x.dtype) * FIELD_SIZE
    idx = (x + offsets[None, :]).astype(jnp.int32).reshape(-1)
    grp = idx >> 3
    fc_flat = fc_table.reshape(-1)
    bias_pad = jnp.broadcast_to(bias.astype(jnp.float32), (L,))
    mesh = plsc.VectorSubcoreMesh(core_axis_name="c", subcore_axis_name="s")
    params = pltpu.CompilerParams(needs_layout_passes=False,
                                  use_tc_tiling_on_sc=True)

    tr = pl.kernel(
        _tr_body,
        out_type=jax.ShapeDtypeStruct((NGRP, GROUP), jnp.float32),
        mesh=mesh,
        compiler_params=params,
        scratch_types=[
            pltpu.VMEM((L, SEG), jnp.float32),
            pltpu.VMEM((SEG_G, GROUP), jnp.float32),
            pltpu.SemaphoreType.DMA,
            pltpu.SemaphoreType.DMA,
        ],
    )
    emb_g = tr(emb_table.T)

    fm = pl.kernel(
        _fm_body,
        out_type=jax.ShapeDtypeStruct((BATCH,), jnp.float32),
        mesh=mesh,
        compiler_params=params,
        scratch_types=[
            pltpu.VMEM((NIDX + L,), jnp.int32),
            pltpu.VMEM((NIDX,), jnp.int32),
            pltpu.VMEM((NIDX, GROUP), jnp.float32),
            pltpu.VMEM((NIDX + L,), jnp.float32),
            pltpu.VMEM((L,), jnp.float32),
            pltpu.VMEM((L,), jnp.float32),
            pltpu.SemaphoreType.DMA,
            pltpu.SemaphoreType.DMA,
        ],
    )
    return fm(idx, grp, emb_g, fc_flat, bias_pad)


# speedup vs baseline: 2.3650x; 1.1509x over previous
import jax
import jax.numpy as jnp
from jax import lax
from jax.experimental import pallas as pl
from jax.experimental.pallas import tpu as pltpu
from jax.experimental.pallas import tpu_sc as plsc

N_FIELDS = 26
EMBED_DIM = 16
FIELD_SIZE = 100000
BATCH = 16384
TOTAL = 2600000
L = 16
NC, NS = 2, 16
NW = NC * NS
ROWS_W = BATCH // NW
CH = 16
NCHUNK = ROWS_W // CH
NIDX = CH * N_FIELDS

SEG = 1024
NSEG = TOTAL // SEG
TAIL = TOTAL - NSEG * SEG
FULL_W = NSEG // NW


def _tr_body(src_hbm, out_hbm, in_v, out_v, sem_i, sem_o):
    wid = lax.axis_index("s") * NC + lax.axis_index("c")
    iota = lax.iota(jnp.int32, L)
    pre = [iota * EMBED_DIM + d for d in range(L)]

    def do_block(r0, width):
        cps = [pltpu.async_copy(src_hbm.at[d, pl.ds(r0, width)],
                                in_v.at[d, pl.ds(0, width)], sem_i)
               for d in range(L)]
        for cp in cps:
            cp.wait()

        def rblk(i, _):
            base = i * (L * EMBED_DIM)
            for d in range(L):
                vals = in_v[d, pl.ds(i * L, L)]
                plsc.store_scatter(out_v, [pre[d] + base], vals)
            return 0

        lax.fori_loop(0, width // L, rblk, 0)
        pltpu.async_copy(out_v.at[pl.ds(0, width * EMBED_DIM)],
                         out_hbm.at[pl.ds(r0 * EMBED_DIM,
                                          width * EMBED_DIM)],
                         sem_o).wait()

    n_w = FULL_W + jnp.where(wid < NSEG - FULL_W * NW, 1, 0)

    def seg_loop(i, _):
        do_block((i * NW + wid) * SEG, SEG)
        return 0

    lax.fori_loop(0, n_w, seg_loop, 0)

    @pl.when(wid == NW - 1)
    def _():
        do_block(NSEG * SEG, TAIL)


def _fm_body(idx_hbm, emb_hbm, fc_hbm, bias_hbm, out_hbm,
             idx_v, emb_v, fc_v, z_v, bias_v, sem_e, sem_f):
    wid = lax.axis_index("s") * NC + lax.axis_index("c")
    pltpu.sync_copy(bias_hbm, bias_v)
    b0 = bias_v[...][0]
    lane = lax.iota(jnp.int32, L)
    mtail = lane < (N_FIELDS - L)

    def chunk(c, _):
        base = wid * (ROWS_W * N_FIELDS) + c * NIDX
        pltpu.sync_copy(idx_hbm.at[pl.ds(base, NIDX)],
                        idx_v.at[pl.ds(0, NIDX)])
        cp_e = pltpu.async_copy(emb_hbm.at[idx_v.at[pl.ds(0, NIDX)]],
                                emb_v, sem_e)
        cp_f = pltpu.async_copy(fc_hbm.at[idx_v.at[pl.ds(0, NIDX)]],
                                fc_v.at[pl.ds(0, NIDX)], sem_f)
        cp_e.wait()
        cp_f.wait()

        def rowi(i, vec):
            rb = i * N_FIELDS
            acc = emb_v[rb, :]
            acc2 = acc * acc
            for f in range(1, N_FIELDS):
                v = emb_v[rb + f, :]
                acc = acc + v
                acc2 = acc2 + v * v
            a = fc_v[pl.ds(rb, L)]
            b = fc_v[pl.ds(rb + L, L)]
            lin = b0 + jnp.sum(a) + jnp.sum(jnp.where(mtail, b, 0.0))
            z = lin + 0.5 * (jnp.sum(acc * acc) - jnp.sum(acc2))
            return jnp.where(lane == i, z, vec)

        vec = lax.fori_loop(0, CH, rowi, jnp.zeros((L,), jnp.float32))
        z_v[...] = 1.0 / (1.0 + jnp.exp(-vec))
        pltpu.sync_copy(z_v, out_hbm.at[pl.ds(wid * ROWS_W + c * CH, CH)])
        return 0

    lax.fori_loop(0, NCHUNK, chunk, 0)


def kernel(x, emb_table, fc_table, bias):
    offsets = jnp.arange(N_FIELDS, dtype=x.dtype) * FIELD_SIZE
    idx = (x + offsets[None, :]).astype(jnp.int32).reshape(-1)
    fc_flat = fc_table.reshape(-1)
    bias_pad = jnp.broadcast_to(bias.astype(jnp.float32), (L,))
    mesh = plsc.VectorSubcoreMesh(core_axis_name="c", subcore_axis_name="s")

    tr = pl.kernel(
        _tr_body,
        out_type=jax.ShapeDtypeStruct((TOTAL * EMBED_DIM,), jnp.float32),
        mesh=mesh,
        compiler_params=pltpu.CompilerParams(needs_layout_passes=False,
                                             use_tc_tiling_on_sc=True),
        scratch_types=[
            pltpu.VMEM((L, SEG), jnp.float32),
            pltpu.VMEM((SEG * EMBED_DIM,), jnp.float32),
            pltpu.SemaphoreType.DMA,
            pltpu.SemaphoreType.DMA,
        ],
    )
    emb_rm = tr(emb_table.T).reshape(TOTAL, EMBED_DIM)

    fm = pl.kernel(
        _fm_body,
        out_type=jax.ShapeDtypeStruct((BATCH,), jnp.float32),
        mesh=mesh,
        compiler_params=pltpu.CompilerParams(needs_layout_passes=False,
                                             use_tc_tiling_on_sc=False),
        scratch_types=[
            pltpu.VMEM((NIDX + L,), jnp.int32),
            pltpu.VMEM((NIDX, EMBED_DIM), jnp.float32),
            pltpu.VMEM((NIDX + L,), jnp.float32),
            pltpu.VMEM((L,), jnp.float32),
            pltpu.VMEM((L,), jnp.float32),
            pltpu.SemaphoreType.DMA,
            pltpu.SemaphoreType.DMA,
        ],
    )
    return fm(idx, emb_rm, fc_flat, bias_pad)


# speedup vs baseline: 3.1455x; 1.3300x over previous
import jax
import jax.numpy as jnp
from jax import lax
from jax.experimental import pallas as pl
from jax.experimental.pallas import tpu as pltpu
from jax.experimental.pallas import tpu_sc as plsc

N_FIELDS = 26
EMBED_DIM = 16
FIELD_SIZE = 100000
BATCH = 16384
TOTAL = 2600000
L = 16
NC, NS = 2, 16
NW = NC * NS
ROWS_W = BATCH // NW
CH = 16
NCHUNK = ROWS_W // CH
NIDX = CH * N_FIELDS

SEG = 1024
NSEG = TOTAL // SEG
TAIL = TOTAL - NSEG * SEG
FULL_W = NSEG // NW


def _tr_body(src_hbm, out_hbm, in0, in1, ou0, ou1, si0, si1, so0, so1):
    wid = lax.axis_index("s") * NC + lax.axis_index("c")
    iota = lax.iota(jnp.int32, L)
    pre = [iota * EMBED_DIM + d for d in range(L)]
    ins, outs = (in0, in1), (ou0, ou1)
    sis, sos = (si0, si1), (so0, so1)

    def blk_r0(j):
        return (j * NW + wid) * SEG

    def start_in(b, j):
        r0 = blk_r0(j)
        for d in range(L):
            pltpu.async_copy(src_hbm.at[d, pl.ds(r0, SEG)],
                             ins[b].at[d], sis[b])

    def wait_in(b):
        pltpu.make_async_copy(src_hbm.at[pl.ds(0, L), pl.ds(0, SEG)],
                              ins[b], sis[b]).wait()

    def start_out(b, j):
        pltpu.async_copy(outs[b],
                         out_hbm.at[pl.ds(blk_r0(j) * EMBED_DIM,
                                          SEG * EMBED_DIM)], sos[b])

    def wait_out(b):
        pltpu.make_async_copy(outs[b],
                              out_hbm.at[pl.ds(0, SEG * EMBED_DIM)],
                              sos[b]).wait()

    def compute(b):
        def rblk(i, _):
            base = i * (L * EMBED_DIM)
            for d in range(L):
                vals = ins[b][d, pl.ds(i * L, L)]
                plsc.store_scatter(outs[b], [pre[d] + base], vals)
            return 0

        lax.fori_loop(0, SEG // L, rblk, 0)

    n_w = FULL_W + jnp.where(wid < NSEG - FULL_W * NW, 1, 0)
    npairs = n_w // 2
    start_in(0, 0)

    def pair(p, _):
        j0 = 2 * p
        j1 = j0 + 1
        wait_in(0)
        start_in(1, j1)
        compute(0)

        @pl.when(p > 0)
        def _():
            wait_out(0)

        start_out(0, j0)

        wait_in(1)

        @pl.when(j1 + 1 < n_w)
        def _():
            start_in(0, j1 + 1)

        compute(1)

        @pl.when(p > 0)
        def _():
            wait_out(1)

        start_out(1, j1)
        return 0

    lax.fori_loop(0, npairs, pair, 0)

    @pl.when(n_w % 2 == 1)
    def _():
        wait_in(0)
        compute(0)
        wait_out(0)
        start_out(0, n_w - 1)

    wait_out(0)
    wait_out(1)

    @pl.when(wid == NW - 1)
    def _():
        r0 = NSEG * SEG
        cps = [pltpu.async_copy(src_hbm.at[d, pl.ds(r0, TAIL)],
                                in0.at[d, pl.ds(0, TAIL)], si0)
               for d in range(L)]
        for cp in cps:
            cp.wait()

        def rblk(i, _):
            base = i * (L * EMBED_DIM)
            for d in range(L):
                vals = in0[d, pl.ds(i * L, L)]
                plsc.store_scatter(ou0, [pre[d] + base], vals)
            return 0

        lax.fori_loop(0, TAIL // L, rblk, 0)
        pltpu.async_copy(ou0.at[pl.ds(0, TAIL * EMBED_DIM)],
                         out_hbm.at[pl.ds(r0 * EMBED_DIM,
                                          TAIL * EMBED_DIM)],
                         so0).wait()


def _fm_body(idx_hbm, emb_hbm, fc_hbm, bias_hbm, out_hbm,
             idx_v, emb_v, fc_v, z_v, bias_v, sem_e, sem_f):
    wid = lax.axis_index("s") * NC + lax.axis_index("c")
    pltpu.sync_copy(bias_hbm, bias_v)
    b0 = bias_v[...][0]
    lane = lax.iota(jnp.int32, L)
    mtail = lane < (N_FIELDS - L)

    def chunk(c, _):
        base = wid * (ROWS_W * N_FIELDS) + c * NIDX
        pltpu.sync_copy(idx_hbm.at[pl.ds(base, NIDX)],
                        idx_v.at[pl.ds(0, NIDX)])
        cp_e = pltpu.async_copy(emb_hbm.at[idx_v.at[pl.ds(0, NIDX)]],
                                emb_v, sem_e)
        cp_f = pltpu.async_copy(fc_hbm.at[idx_v.at[pl.ds(0, NIDX)]],
                                fc_v.at[pl.ds(0, NIDX)], sem_f)
        cp_e.wait()
        cp_f.wait()

        def rowi(i, vec):
            rb = i * N_FIELDS
            acc = emb_v[rb, :]
            acc2 = acc * acc
            for f in range(1, N_FIELDS):
                v = emb_v[rb + f, :]
                acc = acc + v
                acc2 = acc2 + v * v
            a = fc_v[pl.ds(rb, L)]
            b = fc_v[pl.ds(rb + L, L)]
            lin = b0 + jnp.sum(a) + jnp.sum(jnp.where(mtail, b, 0.0))
            z = lin + 0.5 * (jnp.sum(acc * acc) - jnp.sum(acc2))
            return jnp.where(lane == i, z, vec)

        vec = lax.fori_loop(0, CH, rowi, jnp.zeros((L,), jnp.float32))
        z_v[...] = 1.0 / (1.0 + jnp.exp(-vec))
        pltpu.sync_copy(z_v, out_hbm.at[pl.ds(wid * ROWS_W + c * CH, CH)])
        return 0

    lax.fori_loop(0, NCHUNK, chunk, 0)


def kernel(x, emb_table, fc_table, bias):
    offsets = jnp.arange(N_FIELDS, dtype=x.dtype) * FIELD_SIZE
    idx = (x + offsets[None, :]).astype(jnp.int32).reshape(-1)
    fc_flat = fc_table.reshape(-1)
    bias_pad = jnp.broadcast_to(bias.astype(jnp.float32), (L,))
    mesh = plsc.VectorSubcoreMesh(core_axis_name="c", subcore_axis_name="s")

    tr = pl.kernel(
        _tr_body,
        out_type=jax.ShapeDtypeStruct((TOTAL * EMBED_DIM,), jnp.float32),
        mesh=mesh,
        compiler_params=pltpu.CompilerParams(needs_layout_passes=False,
                                             use_tc_tiling_on_sc=True),
        scratch_types=[
            pltpu.VMEM((L, SEG), jnp.float32),
            pltpu.VMEM((L, SEG), jnp.float32),
            pltpu.VMEM((SEG * EMBED_DIM,), jnp.float32),
            pltpu.VMEM((SEG * EMBED_DIM,), jnp.float32),
            pltpu.SemaphoreType.DMA,
            pltpu.SemaphoreType.DMA,
            pltpu.SemaphoreType.DMA,
            pltpu.SemaphoreType.DMA,
        ],
    )
    emb_rm = tr(emb_table.T).reshape(TOTAL, EMBED_DIM)

    fm = pl.kernel(
        _fm_body,
        out_type=jax.ShapeDtypeStruct((BATCH,), jnp.float32),
        mesh=mesh,
        compiler_params=pltpu.CompilerParams(needs_layout_passes=False,
                                             use_tc_tiling_on_sc=False),
        scratch_types=[
            pltpu.VMEM((NIDX + L,), jnp.int32),
            pltpu.VMEM((NIDX, EMBED_DIM), jnp.float32),
            pltpu.VMEM((NIDX + L,), jnp.float32),
            pltpu.VMEM((L,), jnp.float32),
            pltpu.VMEM((L,), jnp.float32),
            pltpu.SemaphoreType.DMA,
            pltpu.SemaphoreType.DMA,
        ],
    )
    return fm(idx, emb_rm, fc_flat, bias_pad)


# speedup vs baseline: 3.4328x; 1.0914x over previous
import jax
import jax.numpy as jnp
from jax import lax
from jax.experimental import pallas as pl
from jax.experimental.pallas import tpu as pltpu
from jax.experimental.pallas import tpu_sc as plsc

N_FIELDS = 26
EMBED_DIM = 16
FIELD_SIZE = 100000
BATCH = 16384
TOTAL = 2600000
L = 16
NC, NS = 2, 16
NW = NC * NS
ROWS_W = BATCH // NW
CH = 16
NCHUNK = ROWS_W // CH
NIDX = CH * N_FIELDS

SEG = 1024
NSEG = TOTAL // SEG
TAIL = TOTAL - NSEG * SEG
FULL_W = NSEG // NW


def _tr_body(src_hbm, out_hbm, in0, in1, ou0, ou1, si0, si1, so0, so1):
    wid = lax.axis_index("s") * NC + lax.axis_index("c")
    iota = lax.iota(jnp.int32, L)
    pre = [iota * EMBED_DIM + d for d in range(L)]
    ins, outs = (in0, in1), (ou0, ou1)
    sis, sos = (si0, si1), (so0, so1)

    def blk_r0(j):
        return (j * NW + wid) * SEG

    def start_in(b, j):
        pltpu.async_copy(src_hbm.at[:, pl.ds(blk_r0(j), SEG)],
                         ins[b], sis[b])

    def wait_in(b):
        pltpu.make_async_copy(src_hbm.at[pl.ds(0, L), pl.ds(0, SEG)],
                              ins[b], sis[b]).wait()

    def start_out(b, j):
        pltpu.async_copy(outs[b],
                         out_hbm.at[pl.ds(blk_r0(j) * EMBED_DIM,
                                          SEG * EMBED_DIM)], sos[b])

    def wait_out(b):
        pltpu.make_async_copy(outs[b],
                              out_hbm.at[pl.ds(0, SEG * EMBED_DIM)],
                              sos[b]).wait()

    def compute(b):
        def rblk(i, _):
            base = i * (L * EMBED_DIM)
            for d in range(L):
                vals = ins[b][d, pl.ds(i * L, L)]
                plsc.store_scatter(outs[b], [pre[d] + base], vals)
            return 0

        lax.fori_loop(0, SEG // L, rblk, 0)

    n_w = FULL_W + jnp.where(wid < NSEG - FULL_W * NW, 1, 0)
    npairs = n_w // 2
    start_in(0, 0)

    def pair(p, _):
        j0 = 2 * p
        j1 = j0 + 1
        wait_in(0)
        start_in(1, j1)
        compute(0)

        @pl.when(p > 0)
        def _():
            wait_out(0)

        start_out(0, j0)

        wait_in(1)

        @pl.when(j1 + 1 < n_w)
        def _():
            start_in(0, j1 + 1)

        compute(1)

        @pl.when(p > 0)
        def _():
            wait_out(1)

        start_out(1, j1)
        return 0

    lax.fori_loop(0, npairs, pair, 0)

    @pl.when(n_w % 2 == 1)
    def _():
        wait_in(0)
        compute(0)
        wait_out(0)
        start_out(0, n_w - 1)

    wait_out(0)
    wait_out(1)

    @pl.when(wid == NW - 1)
    def _():
        r0 = NSEG * SEG
        cps = [pltpu.async_copy(src_hbm.at[d, pl.ds(r0, TAIL)],
                                in0.at[d, pl.ds(0, TAIL)], si0)
               for d in range(L)]
        for cp in cps:
            cp.wait()

        def rblk(i, _):
            base = i * (L * EMBED_DIM)
            for d in range(L):
                vals = in0[d, pl.ds(i * L, L)]
                plsc.store_scatter(ou0, [pre[d] + base], vals)
            return 0

        lax.fori_loop(0, TAIL // L, rblk, 0)
        pltpu.async_copy(ou0.at[pl.ds(0, TAIL * EMBED_DIM)],
                         out_hbm.at[pl.ds(r0 * EMBED_DIM,
                                          TAIL * EMBED_DIM)],
                         so0).wait()


def _fm_body(idx_hbm, emb_hbm, fc_hbm, bias_hbm, out_hbm,
             idx_v, emb_v, fc_v, z_v, bias_v, sem_e, sem_f):
    wid = lax.axis_index("s") * NC + lax.axis_index("c")
    pltpu.sync_copy(bias_hbm, bias_v)
    b0 = bias_v[...][0]
    lane = lax.iota(jnp.int32, L)
    mtail = lane < (N_FIELDS - L)

    def chunk(c, _):
        base = wid * (ROWS_W * N_FIELDS) + c * NIDX
        pltpu.sync_copy(idx_hbm.at[pl.ds(base, NIDX)],
                        idx_v.at[pl.ds(0, NIDX)])
        cp_e = pltpu.async_copy(emb_hbm.at[idx_v.at[pl.ds(0, NIDX)]],
                                emb_v, sem_e)
        cp_f = pltpu.async_copy(fc_hbm.at[idx_v.at[pl.ds(0, NIDX)]],
                                fc_v.at[pl.ds(0, NIDX)], sem_f)
        cp_e.wait()
        cp_f.wait()

        def rowi(i, vec):
            rb = i * N_FIELDS
            acc = emb_v[rb, :]
            acc2 = acc * acc
            for f in range(1, N_FIELDS):
                v = emb_v[rb + f, :]
                acc = acc + v
                acc2 = acc2 + v * v
            a = fc_v[pl.ds(rb, L)]
            b = fc_v[pl.ds(rb + L, L)]
            lin = b0 + jnp.sum(a) + jnp.sum(jnp.where(mtail, b, 0.0))
            z = lin + 0.5 * (jnp.sum(acc * acc) - jnp.sum(acc2))
            return jnp.where(lane == i, z, vec)

        vec = lax.fori_loop(0, CH, rowi, jnp.zeros((L,), jnp.float32))
        z_v[...] = 1.0 / (1.0 + jnp.exp(-vec))
        pltpu.sync_copy(z_v, out_hbm.at[pl.ds(wid * ROWS_W + c * CH, CH)])
        return 0

    lax.fori_loop(0, NCHUNK, chunk, 0)


def kernel(x, emb_table, fc_table, bias):
    offsets = jnp.arange(N_FIELDS, dtype=x.dtype) * FIELD_SIZE
    idx = (x + offsets[None, :]).astype(jnp.int32).reshape(-1)
    fc_flat = fc_table.reshape(-1)
    bias_pad = jnp.broadcast_to(bias.astype(jnp.float32), (L,))
    mesh = plsc.VectorSubcoreMesh(core_axis_name="c", subcore_axis_name="s")

    tr = pl.kernel(
        _tr_body,
        out_type=jax.ShapeDtypeStruct((TOTAL * EMBED_DIM,), jnp.float32),
        mesh=mesh,
        compiler_params=pltpu.CompilerParams(needs_layout_passes=False,
                                             use_tc_tiling_on_sc=True),
        scratch_types=[
            pltpu.VMEM((L, SEG), jnp.float32),
            pltpu.VMEM((L, SEG), jnp.float32),
            pltpu.VMEM((SEG * EMBED_DIM,), jnp.float32),
            pltpu.VMEM((SEG * EMBED_DIM,), jnp.float32),
            pltpu.SemaphoreType.DMA,
            pltpu.SemaphoreType.DMA,
            pltpu.SemaphoreType.DMA,
            pltpu.SemaphoreType.DMA,
        ],
    )
    emb_rm = tr(emb_table.T).reshape(TOTAL, EMBED_DIM)

    fm = pl.kernel(
        _fm_body,
        out_type=jax.ShapeDtypeStruct((BATCH,), jnp.float32),
        mesh=mesh,
        compiler_params=pltpu.CompilerParams(needs_layout_passes=False,
                                             use_tc_tiling_on_sc=False),
        scratch_types=[
            pltpu.VMEM((NIDX + L,), jnp.int32),
            pltpu.VMEM((NIDX, EMBED_DIM), jnp.float32),
            pltpu.VMEM((NIDX + L,), jnp.float32),
            pltpu.VMEM((L,), jnp.float32),
            pltpu.VMEM((L,), jnp.float32),
            pltpu.SemaphoreType.DMA,
            pltpu.SemaphoreType.DMA,
        ],
    )
    return fm(idx, emb_rm, fc_flat, bias_pad)


# speedup vs baseline: 4.4752x; 1.3037x over previous
import jax
import jax.numpy as jnp
from jax import lax
from jax.experimental import pallas as pl
from jax.experimental.pallas import tpu as pltpu
from jax.experimental.pallas import tpu_sc as plsc

N_FIELDS = 26
EMBED_DIM = 16
FIELD_SIZE = 100000
BATCH = 16384
TOTAL = 2600000
L = 16
NC, NS = 2, 16
NW = NC * NS
ROWS_W = BATCH // NW
CH = 16
NCHUNK = ROWS_W // CH
NIDX = CH * N_FIELDS

SEG = 1024
NSEG = TOTAL // SEG
TAIL = TOTAL - NSEG * SEG
FULL_W = NSEG // NW


def _tr_body(src_hbm, out_hbm, in0, in1, ou0, ou1, si0, si1, so0, so1):
    wid = lax.axis_index("s") * NC + lax.axis_index("c")
    iota = lax.iota(jnp.int32, L)
    pre = [iota * EMBED_DIM + d for d in range(L)]
    ins, outs = (in0, in1), (ou0, ou1)
    sis, sos = (si0, si1), (so0, so1)

    def blk_r0(j):
        return (j * NW + wid) * SEG

    def start_in(b, j):
        pltpu.async_copy(src_hbm.at[:, pl.ds(blk_r0(j), SEG)],
                         ins[b], sis[b])

    def wait_in(b):
        pltpu.make_async_copy(src_hbm.at[pl.ds(0, L), pl.ds(0, SEG)],
                              ins[b], sis[b]).wait()

    def start_out(b, j):
        pltpu.async_copy(outs[b],
                         out_hbm.at[pl.ds(blk_r0(j) * EMBED_DIM,
                                          SEG * EMBED_DIM)], sos[b])

    def wait_out(b):
        pltpu.make_async_copy(outs[b],
                              out_hbm.at[pl.ds(0, SEG * EMBED_DIM)],
                              sos[b]).wait()

    def compute(b):
        @plsc.parallel_loop(0, SEG // L, unroll=4)
        def _(i):
            base = i * (L * EMBED_DIM)
            for d in range(L):
                vals = ins[b][d, pl.ds(i * L, L)]
                plsc.store_scatter(outs[b], [pre[d] + base], vals)

    n_w = FULL_W + jnp.where(wid < NSEG - FULL_W * NW, 1, 0)
    npairs = n_w // 2
    start_in(0, 0)

    def pair(p, _):
        j0 = 2 * p
        j1 = j0 + 1
        wait_in(0)
        start_in(1, j1)
        compute(0)

        @pl.when(p > 0)
        def _():
            wait_out(0)

        start_out(0, j0)

        wait_in(1)

        @pl.when(j1 + 1 < n_w)
        def _():
            start_in(0, j1 + 1)

        compute(1)

        @pl.when(p > 0)
        def _():
            wait_out(1)

        start_out(1, j1)
        return 0

    lax.fori_loop(0, npairs, pair, 0)

    @pl.when(n_w % 2 == 1)
    def _():
        wait_in(0)
        compute(0)
        wait_out(0)
        start_out(0, n_w - 1)

    wait_out(0)
    wait_out(1)

    @pl.when(wid == NW - 1)
    def _():
        r0 = NSEG * SEG
        cps = [pltpu.async_copy(src_hbm.at[d, pl.ds(r0, TAIL)],
                                in0.at[d, pl.ds(0, TAIL)], si0)
               for d in range(L)]
        for cp in cps:
            cp.wait()

        def rblk(i, _):
            base = i * (L * EMBED_DIM)
            for d in range(L):
                vals = in0[d, pl.ds(i * L, L)]
                plsc.store_scatter(ou0, [pre[d] + base], vals)
            return 0

        lax.fori_loop(0, TAIL // L, rblk, 0)
        pltpu.async_copy(ou0.at[pl.ds(0, TAIL * EMBED_DIM)],
                         out_hbm.at[pl.ds(r0 * EMBED_DIM,
                                          TAIL * EMBED_DIM)],
                         so0).wait()


def _fm_body(idx_hbm, emb_hbm, fc_hbm, bias_hbm, out_hbm,
             idx_v, emb_v, fc_v, z_v, bias_v, sem_e, sem_f):
    wid = lax.axis_index("s") * NC + lax.axis_index("c")
    pltpu.sync_copy(bias_hbm, bias_v)
    b0 = bias_v[...][0]
    lane = lax.iota(jnp.int32, L)
    mtail = lane < (N_FIELDS - L)

    def chunk(c, _):
        base = wid * (ROWS_W * N_FIELDS) + c * NIDX
        pltpu.sync_copy(idx_hbm.at[pl.ds(base, NIDX)],
                        idx_v.at[pl.ds(0, NIDX)])
        cp_e = pltpu.async_copy(emb_hbm.at[idx_v.at[pl.ds(0, NIDX)]],
                                emb_v, sem_e)
        cp_f = pltpu.async_copy(fc_hbm.at[idx_v.at[pl.ds(0, NIDX)]],
                                fc_v.at[pl.ds(0, NIDX)], sem_f)
        cp_e.wait()
        cp_f.wait()

        def rowi(i, vec):
            rb = i * N_FIELDS
            acc = emb_v[rb, :]
            acc2 = acc * acc
            for f in range(1, N_FIELDS):
                v = emb_v[rb + f, :]
                acc = acc + v
                acc2 = acc2 + v * v
            a = fc_v[pl.ds(rb, L)]
            b = fc_v[pl.ds(rb + L, L)]
            lin = b0 + jnp.sum(a) + jnp.sum(jnp.where(mtail, b, 0.0))
            z = lin + 0.5 * (jnp.sum(acc * acc) - jnp.sum(acc2))
            return jnp.where(lane == i, z, vec)

        vec = lax.fori_loop(0, CH, rowi, jnp.zeros((L,), jnp.float32))
        z_v[...] = 1.0 / (1.0 + jnp.exp(-vec))
        pltpu.sync_copy(z_v, out_hbm.at[pl.ds(wid * ROWS_W + c * CH, CH)])
        return 0

    lax.fori_loop(0, NCHUNK, chunk, 0)


def kernel(x, emb_table, fc_table, bias):
    offsets = jnp.arange(N_FIELDS, dtype=x.dtype) * FIELD_SIZE
    idx = (x + offsets[None, :]).astype(jnp.int32).reshape(-1)
    fc_flat = fc_table.reshape(-1)
    bias_pad = jnp.broadcast_to(bias.astype(jnp.float32), (L,))
    mesh = plsc.VectorSubcoreMesh(core_axis_name="c", subcore_axis_name="s")

    tr = pl.kernel(
        _tr_body,
        out_type=jax.ShapeDtypeStruct((TOTAL * EMBED_DIM,), jnp.float32),
        mesh=mesh,
        compiler_params=pltpu.CompilerParams(needs_layout_passes=False,
                                             use_tc_tiling_on_sc=True),
        scratch_types=[
            pltpu.VMEM((L, SEG), jnp.float32),
            pltpu.VMEM((L, SEG), jnp.float32),
            pltpu.VMEM((SEG * EMBED_DIM,), jnp.float32),
            pltpu.VMEM((SEG * EMBED_DIM,), jnp.float32),
            pltpu.SemaphoreType.DMA,
            pltpu.SemaphoreType.DMA,
            pltpu.SemaphoreType.DMA,
            pltpu.SemaphoreType.DMA,
        ],
    )
    emb_rm = tr(emb_table.T).reshape(TOTAL, EMBED_DIM)

    fm = pl.kernel(
        _fm_body,
        out_type=jax.ShapeDtypeStruct((BATCH,), jnp.float32),
        mesh=mesh,
        compiler_params=pltpu.CompilerParams(needs_layout_passes=False,
                                             use_tc_tiling_on_sc=False),
        scratch_types=[
            pltpu.VMEM((NIDX + L,), jnp.int32),
            pltpu.VMEM((NIDX, EMBED_DIM), jnp.float32),
            pltpu.VMEM((NIDX + L,), jnp.float32),
            pltpu.VMEM((L,), jnp.float32),
            pltpu.VMEM((L,), jnp.float32),
            pltpu.SemaphoreType.DMA,
            pltpu.SemaphoreType.DMA,
        ],
    )
    return fm(idx, emb_rm, fc_flat, bias_pad)
